# baseline (speedup 1.0000x reference)
import functools
import math

import jax
import jax.numpy as jnp
from jax import lax
from jax.experimental import pallas as pl
from jax.experimental.pallas import tpu as pltpu
from jax.experimental.pallas import tpu_sc as plsc

D_MODEL = 64
SCALE = math.sqrt(D_MODEL)
RPG = 8
NGRP = 2
SPLIT = 616880

_info = plsc.get_sparse_core_info()
_NC, _NS, _L = _info.num_cores, _info.num_subcores, _info.num_lanes
_NW = _NC * _NS


@functools.partial(jax.jit, static_argnames=("out_shape", "d"))
def _emb_lookup(x3, table_a3, table_b, out_shape, d):
    b_total = out_shape[0] * out_shape[1]
    b_per_w = b_total // _NW
    n_batch = b_per_w // _L
    och = 128
    nga = table_a3.shape[0]
    ngb = table_b.shape[0] // RPG
    mesh = plsc.VectorSubcoreMesh(core_axis_name="c", subcore_axis_name="s")

    @functools.partial(
        pl.kernel,
        mesh=mesh,
        out_type=jax.ShapeDtypeStruct((*out_shape, d), jnp.float32),
        scratch_types=[
            pltpu.VMEM((RPG, b_per_w // RPG), jnp.int32),
            pltpu.VMEM((NGRP * _L, RPG, d), jnp.float32),
            pltpu.VMEM((NGRP * _L, RPG, d), jnp.float32),
            pltpu.VMEM((och, d), jnp.float32),
            [pltpu.SemaphoreType.DMA] * NGRP,
            [pltpu.SemaphoreType.DMA] * NGRP,
        ],
    )
    def k(idx_hbm, ta_hbm, tb_hbm, out_hbm, idx_v, bufa_v, bufb_v, stage_v,
          sems_a, sems_b):
        wid = lax.axis_index("s") * _NC + lax.axis_index("c")
        base = wid * b_per_w
        out2 = out_hbm.reshape(b_total, d)
        tb3 = tb_hbm.reshape(ngb, RPG, d)
        pltpu.sync_copy(idx_hbm.at[wid], idx_v)
        ncol = b_per_w // RPG

        def batch_vec(bb):
            j = (bb * _L) // ncol
            col = (bb * _L) % ncol
            return idx_v[j, pl.ds(col, _L)]

        def fire(bb, u):
            iv = batch_vec(bb)
            ga = jnp.minimum(iv, SPLIT - 1) >> 3
            gb = jnp.minimum(jnp.maximum(iv - SPLIT, 0) >> 3, ngb - 1)
            for s in range(_L):
                pltpu.async_copy(ta_hbm.at[ga[s]], bufa_v.at[u * _L + s], sems_a[u])
            for s in range(_L):
                pltpu.async_copy(tb3.at[gb[s]], bufb_v.at[u * _L + s], sems_b[u])

        def drain(u):
            pltpu.make_async_copy(
                ta_hbm.at[pl.ds(0, _L)], bufa_v.at[pl.ds(0, _L)], sems_a[u]
            ).wait()
            pltpu.make_async_copy(
                tb3.at[pl.ds(0, _L)], bufb_v.at[pl.ds(0, _L)], sems_b[u]
            ).wait()

        def select(bb, u):
            iv = batch_vec(bb)
            ov = iv & 7
            sa = jnp.where(iv < SPLIT, jnp.float32(SCALE), 0.0)
            sb = jnp.where(iv < SPLIT, 0.0, jnp.float32(SCALE))
            for s in range(_L):
                srow = ((bb * _L) % och) + s
                for kk in range(d // _L):
                    sl = pl.ds(kk * _L, _L)
                    stage_v[srow, sl] = (
                        bufa_v[u * _L + s, ov[s], sl] * sa[s]
                        + bufb_v[u * _L + s, ov[s], sl] * sb[s]
                    )

        for u in range(NGRP):
            fire(u, u)

        def body(t, carry):
            for u in range(NGRP):
                bb = NGRP * t + u
                drain(u)
                select(bb, u)

                @pl.when(t < n_batch // NGRP - 1)
                def _():
                    fire(bb + NGRP, u)

            @pl.when((t & 3) == 3)
            def _():
                pltpu.sync_copy(
                    stage_v, out2.at[pl.ds(base + (t >> 2) * och, och)]
                )

            return carry

        lax.fori_loop(0, n_batch // NGRP, body, 0)

    return k(x3, table_a3, table_b)


def kernel(x, weight):
    b0, b1 = x.shape
    b_total = b0 * b1
    b_per_w = b_total // _NW
    x3 = x.astype(jnp.int32).reshape(_NW, RPG, b_per_w // RPG)
    table_a3 = weight[:SPLIT].reshape(SPLIT // RPG, RPG, D_MODEL)
    table_b = weight[SPLIT:]
    return _emb_lookup(x3, table_a3, table_b, (b0, b1), D_MODEL)


# speedup vs baseline: 3.1285x; 3.1285x over previous
import functools
import math

import jax
import jax.numpy as jnp
from jax import lax
from jax.experimental import pallas as pl
from jax.experimental.pallas import tpu as pltpu
from jax.experimental.pallas import tpu_sc as plsc

D_MODEL = 64
SCALE = math.sqrt(D_MODEL)
RPG = 8
NGRP = 4

_info = plsc.get_sparse_core_info()
_NC, _NS, _L = _info.num_cores, _info.num_subcores, _info.num_lanes
_NW = _NC * _NS


@functools.partial(jax.jit, static_argnames=("out_shape", "d"))
def _emb_lookup(x3, table3, out_shape, d):
    b_total = out_shape[0] * out_shape[1]
    b_per_w = b_total // _NW
    n_batch = b_per_w // _L
    och = 128
    mesh = plsc.VectorSubcoreMesh(core_axis_name="c", subcore_axis_name="s")

    @functools.partial(
        pl.kernel,
        mesh=mesh,
        out_type=jax.ShapeDtypeStruct((*out_shape, d), jnp.float32),
        scratch_types=[
            pltpu.VMEM((RPG, b_per_w // RPG), jnp.int32),
            pltpu.VMEM((NGRP * _L, RPG, d), jnp.float32),
            pltpu.VMEM((och, d), jnp.float32),
            [pltpu.SemaphoreType.DMA] * NGRP,
        ],
    )
    def k(idx_hbm, tab_hbm, out_hbm, idx_v, buf_v, stage_v, sems):
        wid = lax.axis_index("s") * _NC + lax.axis_index("c")
        base = wid * b_per_w
        out2 = out_hbm.reshape(b_total, d)
        pltpu.sync_copy(idx_hbm.at[wid], idx_v)
        ncol = b_per_w // RPG

        def batch_vec(bb):
            j = (bb * _L) // ncol
            col = (bb * _L) % ncol
            return idx_v[j, pl.ds(col, _L)]

        def fire(bb, u):
            gv = batch_vec(bb) >> 3
            for s in range(_L):
                pltpu.async_copy(tab_hbm.at[gv[s]], buf_v.at[u * _L + s], sems[u])

        def drain(u):
            pltpu.make_async_copy(
                tab_hbm.at[pl.ds(0, _L)], buf_v.at[pl.ds(0, _L)], sems[u]
            ).wait()

        def select(bb, u):
            ov = batch_vec(bb) & 7
            for s in range(_L):
                srow = ((bb * _L) % och) + s
                for kk in range(d // _L):
                    sl = pl.ds(kk * _L, _L)
                    stage_v[srow, sl] = buf_v[u * _L + s, ov[s], sl] * SCALE

        for u in range(NGRP):
            fire(u, u)

        def body(t, carry):
            for u in range(NGRP):
                bb = NGRP * t + u
                drain(u)
                select(bb, u)

                @pl.when(t < n_batch // NGRP - 1)
                def _():
                    fire(bb + NGRP, u)

            @pl.when((t & 1) == 1)
            def _():
                pltpu.sync_copy(
                    stage_v, out2.at[pl.ds(base + (t >> 1) * och, och)]
                )

            return carry

        lax.fori_loop(0, n_batch // NGRP, body, 0)

    return k(x3, table3)


def kernel(x, weight):
    b0, b1 = x.shape
    b_total = b0 * b1
    b_per_w = b_total // _NW
    x3 = x.astype(jnp.int32).reshape(_NW, RPG, b_per_w // RPG)
    table3 = weight.reshape(weight.shape[0] // RPG, RPG, D_MODEL)
    return _emb_lookup(x3, table3, (b0, b1), D_MODEL)


# speedup vs baseline: 3.1340x; 1.0017x over previous
import functools
import math

import jax
import jax.numpy as jnp
from jax import lax
from jax.experimental import pallas as pl
from jax.experimental.pallas import tpu as pltpu
from jax.experimental.pallas import tpu_sc as plsc

D_MODEL = 64
SCALE = math.sqrt(D_MODEL)
RPG = 8
NGRP = 4

_info = plsc.get_sparse_core_info()
_NC, _NS, _L = _info.num_cores, _info.num_subcores, _info.num_lanes
_NW = _NC * _NS


@functools.partial(jax.jit, static_argnames=("out_shape", "d"))
def _emb_lookup(x3, table3, out_shape, d):
    b_total = out_shape[0] * out_shape[1]
    b_per_w = b_total // _NW
    n_batch = b_per_w // _L
    och = 256
    mesh = plsc.VectorSubcoreMesh(core_axis_name="c", subcore_axis_name="s")

    @functools.partial(
        pl.kernel,
        mesh=mesh,
        out_type=jax.ShapeDtypeStruct((*out_shape, d), jnp.float32),
        scratch_types=[
            pltpu.VMEM((RPG, b_per_w // RPG), jnp.int32),
            pltpu.VMEM((NGRP * _L, RPG, d), jnp.float32),
            pltpu.VMEM((och, d), jnp.float32),
            [pltpu.SemaphoreType.DMA] * NGRP,
        ],
    )
    def k(idx_hbm, tab_hbm, out_hbm, idx_v, buf_v, stage_v, sems):
        wid = lax.axis_index("s") * _NC + lax.axis_index("c")
        base = wid * b_per_w
        out2 = out_hbm.reshape(b_total, d)
        pltpu.sync_copy(idx_hbm.at[wid], idx_v)
        ncol = b_per_w // RPG

        def batch_vec(bb):
            j = (bb * _L) // ncol
            col = (bb * _L) % ncol
            return idx_v[j, pl.ds(col, _L)]

        def fire(bb, u):
            gv = batch_vec(bb) >> 3
            for s in range(_L):
                pltpu.async_copy(tab_hbm.at[gv[s]], buf_v.at[u * _L + s], sems[u])

        def drain(u):
            pltpu.make_async_copy(
                tab_hbm.at[pl.ds(0, _L)], buf_v.at[pl.ds(0, _L)], sems[u]
            ).wait()

        def select(bb, u):
            ov = batch_vec(bb) & 7
            for s in range(_L):
                srow = ((bb * _L) % och) + s
                for kk in range(d // _L):
                    sl = pl.ds(kk * _L, _L)
                    stage_v[srow, sl] = buf_v[u * _L + s, ov[s], sl] * SCALE

        for u in range(NGRP):
            fire(u, u)

        def body(t, carry):
            for u in range(NGRP):
                bb = NGRP * t + u
                drain(u)
                select(bb, u)

                @pl.when(t < n_batch // NGRP - 1)
                def _():
                    fire(bb + NGRP, u)

            @pl.when((t & 3) == 3)
            def _():
                pltpu.sync_copy(
                    stage_v, out2.at[pl.ds(base + (t >> 2) * och, och)]
                )

            return carry

        lax.fori_loop(0, n_batch // NGRP, body, 0)

    return k(x3, table3)


def kernel(x, weight):
    b0, b1 = x.shape
    b_total = b0 * b1
    b_per_w = b_total // _NW
    x3 = x.astype(jnp.int32).reshape(_NW, RPG, b_per_w // RPG)
    table3 = weight.reshape(weight.shape[0] // RPG, RPG, D_MODEL)
    return _emb_lookup(x3, table3, (b0, b1), D_MODEL)
